# baseline (speedup 1.0000x reference)
import functools

import jax
import jax.numpy as jnp
from jax import lax
from jax.experimental import pallas as pl
from jax.experimental.pallas import tpu as pltpu
from jax.experimental.pallas import tpu_sc as plsc

_NC = 2
_NS = 16
_NW = _NC * _NS
_L = 16

_ROWS = 3072
_COLS = 4096
_ROWS_W = _ROWS // _NW
_SR = 8
_NCH = _ROWS_W // _SR
_NB = 3
_LUT_N = 4096


def _sc_body(x_hbm, lut_hbm, out_hbm, lut_v, v0, v1, v2, s0, s1, s2):
    wid = lax.axis_index("s") * _NC + lax.axis_index("c")
    base_row = wid * _ROWS_W
    bufs = (v0, v1, v2)
    sems = (s0, s1, s2)

    def start_in(c, b):
        r0 = base_row + c * _SR
        pltpu.async_copy(x_hbm.at[pl.ds(r0, _SR), :], bufs[b], sems[b])

    def start_out(c, b):
        r0 = base_row + c * _SR
        pltpu.async_copy(bufs[b].bitcast(jnp.float32),
                         out_hbm.at[pl.ds(r0, _SR), :], sems[b])

    def drain_in(b):
        pltpu.make_async_copy(x_hbm.at[pl.ds(0, _SR), :], bufs[b],
                              sems[b]).wait()

    def drain_out(b):
        pltpu.make_async_copy(bufs[b].bitcast(jnp.float32),
                              out_hbm.at[pl.ds(0, _SR), :], sems[b]).wait()

    start_in(0, 0)
    pltpu.sync_copy(lut_hbm, lut_v)

    @pl.loop(0, _NCH, step=_NB)
    def chunk_body(c0):
        for b in range(_NB):
            c = c0 + b
            bn = (b + 1) % _NB

            @pl.when(c >= 2)
            def _():
                drain_out(bn)

            @pl.when(c + 1 < _NCH)
            def _():
                start_in(c + 1, bn)

            drain_in(b)

            @plsc.parallel_loop(0, _COLS, step=_L, unroll=2)
            def gather_step(i):
                for r in range(_SR):
                    idx = bufs[b][r, pl.ds(i, _L)] & (_LUT_N - 1)
                    val = plsc.load_gather(lut_v, [idx])
                    bufs[b][r, pl.ds(i, _L)] = plsc.bitcast(val, jnp.int32)

            start_out(c, b)

    drain_out((_NCH - 2) % _NB)
    drain_out((_NCH - 1) % _NB)


_sc_kernel = functools.partial(
    pl.kernel,
    mesh=plsc.VectorSubcoreMesh(core_axis_name="c", subcore_axis_name="s"),
    out_type=jax.ShapeDtypeStruct((_ROWS, _COLS), jnp.float32),
    scratch_types=[
        pltpu.VMEM((_LUT_N,), jnp.float32),
        pltpu.VMEM((_SR, _COLS), jnp.int32),
        pltpu.VMEM((_SR, _COLS), jnp.int32),
        pltpu.VMEM((_SR, _COLS), jnp.int32),
        pltpu.SemaphoreType.DMA,
        pltpu.SemaphoreType.DMA,
        pltpu.SemaphoreType.DMA,
    ],
    compiler_params=pltpu.CompilerParams(needs_layout_passes=False),
)(_sc_body)


def kernel(x, lut):
    return _sc_kernel(x, lut)


# speedup vs baseline: 1.0159x; 1.0159x over previous
import functools

import jax
import jax.numpy as jnp
from jax import lax
from jax.experimental import pallas as pl
from jax.experimental.pallas import tpu as pltpu
from jax.experimental.pallas import tpu_sc as plsc

_NC = 2
_NS = 16
_NW = _NC * _NS
_L = 16

_ROWS = 3072
_COLS = 4096
_ROWS_W = _ROWS // _NW
_SR = 8
_NCH = _ROWS_W // _SR
_NB = 3
_LUT_N = 4096


def _sc_body(x_hbm, lut_hbm, out_hbm, lut_v, v0, v1, v2, s0, s1, s2):
    wid = lax.axis_index("s") * _NC + lax.axis_index("c")
    base_row = wid * _ROWS_W
    bufs = (v0, v1, v2)
    sems = (s0, s1, s2)

    def start_in(c, b):
        r0 = base_row + c * _SR
        pltpu.async_copy(x_hbm.at[pl.ds(r0, _SR), :], bufs[b], sems[b])

    def start_out(c, b):
        r0 = base_row + c * _SR
        pltpu.async_copy(bufs[b].bitcast(jnp.float32),
                         out_hbm.at[pl.ds(r0, _SR), :], sems[b])

    def drain_in(b):
        pltpu.make_async_copy(x_hbm.at[pl.ds(0, _SR), :], bufs[b],
                              sems[b]).wait()

    def drain_out(b):
        pltpu.make_async_copy(bufs[b].bitcast(jnp.float32),
                              out_hbm.at[pl.ds(0, _SR), :], sems[b]).wait()

    start_in(0, 0)
    pltpu.sync_copy(lut_hbm, lut_v)

    @pl.loop(0, _NCH, step=_NB)
    def chunk_body(c0):
        for b in range(_NB):
            c = c0 + b
            bn = (b + 1) % _NB

            @pl.when(c >= 2)
            def _():
                drain_out(bn)

            @pl.when(c + 1 < _NCH)
            def _():
                start_in(c + 1, bn)

            drain_in(b)

            @plsc.parallel_loop(0, _COLS, step=_L, unroll=2)
            def gather_step(i):
                for r in range(_SR):
                    idx = bufs[b][r, pl.ds(i, _L)]
                    val = plsc.load_gather(lut_v, [idx])
                    bufs[b][r, pl.ds(i, _L)] = plsc.bitcast(val, jnp.int32)

            start_out(c, b)

    drain_out((_NCH - 2) % _NB)
    drain_out((_NCH - 1) % _NB)


_sc_kernel = functools.partial(
    pl.kernel,
    mesh=plsc.VectorSubcoreMesh(core_axis_name="c", subcore_axis_name="s"),
    out_type=jax.ShapeDtypeStruct((_ROWS, _COLS), jnp.float32),
    scratch_types=[
        pltpu.VMEM((_LUT_N,), jnp.float32),
        pltpu.VMEM((_SR, _COLS), jnp.int32),
        pltpu.VMEM((_SR, _COLS), jnp.int32),
        pltpu.VMEM((_SR, _COLS), jnp.int32),
        pltpu.SemaphoreType.DMA,
        pltpu.SemaphoreType.DMA,
        pltpu.SemaphoreType.DMA,
    ],
    compiler_params=pltpu.CompilerParams(needs_layout_passes=False),
)(_sc_body)


def kernel(x, lut):
    return _sc_kernel(x, lut)
